# Initial kernel scaffold; baseline (speedup 1.0000x reference)
import jax
import jax.numpy as jnp
from jax.experimental import pallas as pl


def kernel(x, edge_index, lambda_max, W2, b2, W4, b4, W6, b6):
    raise NotImplementedError("write your pallas kernel here")


# speedup vs baseline: 11.8603x; 11.8603x over previous
import jax
import jax.numpy as jnp
from jax import lax
from jax.experimental import pallas as pl
from jax.experimental.pallas import tpu as pltpu
from jax.experimental.pallas import tpu_sc as plsc

N = 10000
NP = 10240
RT = 640
RC = 32
E = 320000
EC = 64
BB = 16
CPT = 320
NB = CPT // BB
NCHP = 16 * CPT
NPADR = NP - N
H = 64
F32 = jnp.float32
I32 = jnp.int32


def _vsplat(ref, idx):
    return plsc.load_gather(ref, [jnp.full((16,), idx, I32)])


def _sc_body(x_cat, src2d, dst2d, coeff, t_all,
             u_sp, agg,
             srcb, dstb, gA, gB, hp, hpp, ab,
             deg_t, dis_t, g1t, coeff_v,
             semA, semB):
    c = lax.axis_index("c")
    s = lax.axis_index("s")
    rb0 = s * RT
    ebase = s * CPT
    zero16 = jnp.zeros((16,), F32)
    one16 = jnp.ones((16,), F32)
    iota16 = lax.iota(I32, 16)

    def _fill(buf, nrows, vec):
        def _body(i, _):
            for f in range(4):
                buf[i, pl.ds(f * 16, 16)] = vec
            return 0
        lax.fori_loop(0, nrows, _body, 0, unroll=4)

    _fill(gA, EC, one16)
    _fill(gB, EC, zero16)
    for r in range(RT // EC):
        pltpu.sync_copy(gB, agg.at[pl.ds(rb0 + r * EC, EC)])
    pltpu.sync_copy(coeff, coeff_v)

    plsc.subcore_barrier()

    def _histb(b, _):
        pltpu.sync_copy(src2d.at[pl.ds(ebase + b * BB, BB)], srcb)

        def _hist(j, _):
            pltpu.sync_copy(gA, agg.at[srcb.at[j]], add=True)
            return 0
        lax.fori_loop(0, BB, _hist, 0)
        return 0
    lax.fori_loop(0, NB, _histb, 0)

    plsc.subcore_barrier()

    for r in range(RT // RC):
        pltpu.sync_copy(agg.at[pl.ds(rb0 + r * RC, RC)], ab)

        def _dx_body(i, _):
            deg_t[pl.ds(r * RC + i * 16, 16)] = plsc.load_gather(
                ab, [i * 16 + iota16, jnp.zeros((16,), I32)])
            return 0
        lax.fori_loop(0, RC // 16, _dx_body, 0)
        pltpu.sync_copy(gB.at[pl.ds(0, RC)], agg.at[pl.ds(rb0 + r * RC, RC)])

    c0v = coeff_v[0, pl.ds(0, 16)]

    def _dis_body(i, _):
        d = deg_t[pl.ds(i * 16, 16)]
        ib = lax.bitcast_convert_type(d, I32)
        y = lax.bitcast_convert_type(jnp.int32(0x5F3759DF) - (ib >> 1), F32)
        for _ in range(3):
            y = y * (1.5 - 0.5 * d * y * y)
        dis = jnp.where(d > 0.5, y, 0.0).astype(F32)
        dis_t[pl.ds(i * 16, 16)] = dis
        g1t[pl.ds(i * 16, 16)] = -(c0v * dis)
        return 0
    lax.fori_loop(0, RT // 16, _dis_body, 0)

    for r in range(RT // RC):
        rb = rb0 + r * RC
        pltpu.sync_copy(x_cat.at[c, pl.ds(rb, RC)], hp)
        pltpu.sync_copy(hp, t_all.at[0, c, pl.ds(rb, RC)])

        def _u0_body(i, _):
            vd = _vsplat(dis_t, r * RC + i)
            for f in range(4):
                hpp[i, pl.ds(f * 16, 16)] = vd * hp[i, pl.ds(f * 16, 16)]
            return 0
        lax.fori_loop(0, RC, _u0_body, 0)
        pltpu.sync_copy(hpp.at[pl.ds(0, RC)], u_sp.at[pl.ds(rb, RC)])

    plsc.subcore_barrier()

    def _prop(k, _):
        km1 = k - 1
        km2 = jnp.maximum(k - 2, 0)
        scale = jnp.where(k == 1, 1.0, 2.0).astype(F32)
        va16 = scale * coeff_v[1, pl.ds(0, 16)]
        beta16 = jnp.where(k == 1, 0.0, -1.0).astype(F32) * one16
        vs16 = scale * one16

        def _batch(b, _):
            pltpu.sync_copy(src2d.at[pl.ds(ebase + b * BB, BB)], srcb)
            pltpu.sync_copy(dst2d.at[pl.ds(ebase + b * BB, BB)], dstb)
            pltpu.async_copy(u_sp.at[srcb.at[0]], gA, semA)

            def _pair(p, _):
                j0 = 2 * p
                j1 = j0 + 1
                pltpu.async_copy(u_sp.at[srcb.at[j1]], gB, semB)
                pltpu.make_async_copy(u_sp.at[srcb.at[j0]], gA, semA).wait()
                pltpu.sync_copy(gA, agg.at[dstb.at[j0]], add=True)

                @pl.when(j0 + 2 < BB)
                def _():
                    pltpu.async_copy(u_sp.at[srcb.at[j0 + 2]], gA, semA)
                pltpu.make_async_copy(u_sp.at[srcb.at[j1]], gB, semB).wait()
                pltpu.sync_copy(gB, agg.at[dstb.at[j1]], add=True)
                return 0
            lax.fori_loop(0, BB // 2, _pair, 0)
            return 0
        lax.fori_loop(0, NB, _batch, 0)

        plsc.subcore_barrier()

        _fill(gB, EC, zero16)

        for r in range(RT // RC):
            rb = rb0 + r * RC
            pltpu.sync_copy(agg.at[pl.ds(rb, RC)], ab)
            pltpu.sync_copy(t_all.at[km1, c, pl.ds(rb, RC)], hp)
            pltpu.sync_copy(t_all.at[km2, c, pl.ds(rb, RC)], hpp)

            def _row(i, _):
                li = r * RC + i
                vg = vs16 * _vsplat(g1t, li)
                vd = _vsplat(dis_t, li)
                for f in range(4):
                    sl = pl.ds(f * 16, 16)
                    t = (va16 * hp[i, sl] + beta16 * hpp[i, sl]
                         + vg * ab[i, sl])
                    hp[i, sl] = t
                    hpp[i, sl] = vd * t
                return 0
            lax.fori_loop(0, RC, _row, 0)

            pltpu.sync_copy(hp, t_all.at[k, c, pl.ds(rb, RC)])
            pltpu.sync_copy(hpp, u_sp.at[pl.ds(rb, RC)])
            pltpu.sync_copy(gB.at[pl.ds(0, RC)], agg.at[pl.ds(rb, RC)])

        plsc.subcore_barrier()
        return 0

    lax.fori_loop(1, 6, _prop, 0)


def _make_sc():
    mesh = plsc.VectorSubcoreMesh(core_axis_name="c", subcore_axis_name="s",
                                  num_cores=2, num_subcores=16)
    return pl.kernel(
        _sc_body,
        out_type=jax.ShapeDtypeStruct((6, 2, NP, H), F32),
        mesh=mesh,
        compiler_params=pltpu.CompilerParams(needs_layout_passes=False),
        scratch_types=[
            pltpu.VMEM_SHARED((NP, H), F32),
            pltpu.VMEM_SHARED((NP, H), F32),
            pltpu.VMEM((BB, EC), I32),
            pltpu.VMEM((BB, EC), I32),
            pltpu.VMEM((EC, H), F32),
            pltpu.VMEM((EC, H), F32),
            pltpu.VMEM((RC, H), F32),
            pltpu.VMEM((RC, H), F32),
            pltpu.VMEM((RC, H), F32),
            pltpu.VMEM((RT,), F32),
            pltpu.VMEM((RT,), F32),
            pltpu.VMEM((RT,), F32),
            pltpu.VMEM((2, 16), F32),
            pltpu.SemaphoreType.DMA,
            pltpu.SemaphoreType.DMA,
        ],
    )


BM = 512


def _mm_body(tref, w, b, o):
    acc = jnp.broadcast_to(b[0, :], (BM, 384)).astype(F32)
    for k in range(6):
        acc = acc + jnp.dot(tref[k, 0], w[2 * k], preferred_element_type=F32)
        acc = acc + jnp.dot(tref[k, 1], w[2 * k + 1],
                            preferred_element_type=F32)
    o[...] = acc


def _make_mm():
    nb = NP // BM
    return pl.pallas_call(
        _mm_body,
        grid=(nb,),
        in_specs=[
            pl.BlockSpec((6, 2, BM, H), lambda i: (0, 0, i, 0)),
            pl.BlockSpec((12, H, 384), lambda i: (0, 0, 0)),
            pl.BlockSpec((1, 384), lambda i: (0, 0)),
        ],
        out_specs=pl.BlockSpec((BM, 384), lambda i: (i, 0)),
        out_shape=jax.ShapeDtypeStruct((NP, 384), F32),
    )


def kernel(x, edge_index, lambda_max, W2, b2, W4, b4, W6, b6):
    lam = jnp.asarray(lambda_max, F32)
    c0 = 2.0 / lam
    c1 = c0 - 1.0
    coeff = jnp.stack([jnp.full((16,), c0), jnp.full((16,), c1)]).astype(F32)

    src = edge_index[0].astype(I32)
    dst = edge_index[1].astype(I32)
    eid = jnp.arange(E, dtype=I32) % jnp.int32(NPADR // 2)
    selfm = src == dst
    src = jnp.where(selfm, N + 2 * eid, src)
    dst = jnp.where(selfm, N + 2 * eid + 1, dst)
    npad = NCHP * EC - E
    pid = jnp.arange(npad, dtype=I32) % jnp.int32(NPADR // 2)
    src2d = jnp.concatenate([src, N + 2 * pid]).reshape(NCHP, EC)
    dst2d = jnp.concatenate([dst, N + 2 * pid + 1]).reshape(NCHP, EC)

    pad = jnp.zeros((NP - N, 128), F32)
    xp = jnp.concatenate([x, pad], axis=0)
    x_cat = jnp.stack([xp[:, :H], xp[:, H:]])

    t_all = _make_sc()(x_cat, src2d, dst2d, coeff)

    Wf = jnp.zeros((6, 128, 384), F32)
    Wf = Wf.at[:2, :, 0:128].set(W2)
    Wf = Wf.at[:4, :, 128:256].set(W4)
    Wf = Wf.at[:, :, 256:384].set(W6)
    Wstack = Wf.reshape(12, H, 384)
    bcat = jnp.concatenate([b2, b4, b6]).reshape(1, 384)

    out = _make_mm()(t_all, Wstack, bcat)
    return out[:N]
